# baseline (speedup 1.0000x reference)
import functools

import jax
import jax.numpy as jnp
from jax import lax
from jax.experimental import pallas as pl
from jax.experimental.pallas import tpu as pltpu
from jax.experimental.pallas import tpu_sc as plsc

NC = 2
NS = 16
NW = NC * NS
LANES = 16

VOCAB = 1000000
BATCH = 4096
SEQ = 200
EMBED_DIM = 64

BPW = BATCH // NW
NV = EMBED_DIM // LANES
INV_SEQ = 1.0 / SEQ

BR = 2048
NBLK_IN = -(-VOCAB // BR)
NBLK_OUT = -(-NBLK_IN // 2)
RROWS = NBLK_OUT * BR

CHUNK = 128
NCHUNK_W = BPW * SEQ // CHUNK


def _repack_tc(table_t):
    def rep(lo_ref, hi_ref, o_ref):
        o_ref[...] = jnp.concatenate(
            [
                jnp.swapaxes(lo_ref[...], 0, 1),
                jnp.swapaxes(hi_ref[...], 0, 1),
            ],
            axis=1,
        )

    return pl.pallas_call(
        rep,
        grid=(NBLK_OUT,),
        in_specs=[
            pl.BlockSpec((EMBED_DIM, BR), lambda i: (0, 2 * i)),
            pl.BlockSpec(
                (EMBED_DIM, BR),
                lambda i: (0, jnp.minimum(2 * i + 1, NBLK_IN - 1)),
            ),
        ],
        out_specs=pl.BlockSpec((BR, 128), lambda i: (i, 0)),
        out_shape=jax.ShapeDtypeStruct((RROWS, 128), jnp.float32),
    )(table_t, table_t)


def _pool_sc(r64, gidx):
    mesh = plsc.VectorSubcoreMesh(core_axis_name="c", subcore_axis_name="s")

    @functools.partial(
        pl.kernel,
        mesh=mesh,
        compiler_params=pltpu.CompilerParams(use_tc_tiling_on_sc=False),
        out_type=jax.ShapeDtypeStruct((BATCH, EMBED_DIM), jnp.float32),
        scratch_types=[
            pltpu.VMEM((NCHUNK_W, CHUNK), jnp.int32),
            pltpu.VMEM((2, CHUNK, EMBED_DIM), jnp.float32),
            pltpu.VMEM((BPW, EMBED_DIM), jnp.float32),
            pltpu.SemaphoreType.DMA,
            pltpu.SemaphoreType.DMA,
        ],
    )
    def kb(r_hbm, idx_hbm, out_hbm, idx_v, rows_v, pooled_v, sem0, sem1):
        wid = lax.axis_index("s") * NC + lax.axis_index("c")
        cbase = wid * NCHUNK_W
        pltpu.sync_copy(idx_hbm.at[pl.ds(cbase, NCHUNK_W)], idx_v)

        def start(c, slot, sem):
            pltpu.async_copy(r_hbm.at[idx_v.at[c]], rows_v.at[slot], sem)

        def wait(c, slot, sem):
            pltpu.make_async_copy(
                r_hbm.at[idx_v.at[c]], rows_v.at[slot], sem
            ).wait()

        def reduce_chunk(c, slot, accs):
            s_raw = (SEQ - (CHUNK * c) % SEQ) % SEQ
            s = jnp.minimum(s_raw, CHUNK)
            flush = jnp.logical_and(s_raw < CHUNK,
                                    jnp.logical_or(c > 0, s_raw > 0))

            def red(rr, a):
                return tuple(
                    a[j] + rows_v[slot, rr, pl.ds(16 * j, 16)]
                    for j in range(NV)
                )

            accs = lax.fori_loop(jnp.int32(0), s, red, accs)

            @pl.when(flush)
            def _():
                rf = (CHUNK * c + s - 1) // SEQ
                for j in range(NV):
                    pooled_v[rf, pl.ds(16 * j, 16)] = accs[j] * INV_SEQ

            accs = tuple(
                jnp.where(flush, jnp.zeros((LANES,), jnp.float32), a)
                for a in accs
            )
            return lax.fori_loop(s, jnp.int32(CHUNK), red, accs)

        start(0, 0, sem0)

        def body(cp, accs):
            c0 = 2 * cp
            start(c0 + 1, 1, sem1)
            wait(c0, 0, sem0)
            accs = reduce_chunk(c0, 0, accs)

            @pl.when(c0 + 2 < NCHUNK_W)
            def _():
                start(c0 + 2, 0, sem0)

            wait(c0 + 1, 1, sem1)
            return reduce_chunk(c0 + 1, 1, accs)

        accs = lax.fori_loop(
            0, NCHUNK_W // 2, body,
            tuple(jnp.zeros((LANES,), jnp.float32) for _ in range(NV)),
        )
        for j in range(NV):
            pooled_v[BPW - 1, pl.ds(16 * j, 16)] = accs[j] * INV_SEQ
        pltpu.sync_copy(pooled_v, out_hbm.at[pl.ds(wid * BPW, BPW)])

    return kb(r64, gidx)


def _classifier_tc(pooled, W, b2):
    B, D = pooled.shape
    C = W.shape[0]
    BM = 512

    def mm(x_ref, w_ref, b_ref, o_ref):
        o_ref[...] = (
            lax.dot_general(
                x_ref[...],
                w_ref[...],
                (((1,), (1,)), ((), ())),
                preferred_element_type=jnp.float32,
            )
            + b_ref[...]
        )

    return pl.pallas_call(
        mm,
        grid=(B // BM,),
        in_specs=[
            pl.BlockSpec((BM, D), lambda i: (i, 0)),
            pl.BlockSpec((C, D), lambda i: (0, 0)),
            pl.BlockSpec((1, C), lambda i: (0, 0)),
        ],
        out_specs=pl.BlockSpec((BM, C), lambda i: (i, 0)),
        out_shape=jax.ShapeDtypeStruct((B, C), jnp.float32),
    )(pooled, W, b2)


def kernel(x_data, table, W, b):
    x = x_data.astype(jnp.int32)
    gidx = x.reshape(BATCH * SEQ // CHUNK, CHUNK)
    r2 = lax.optimization_barrier(table.reshape(VOCAB // 2, 128))
    r64 = r2.reshape(VOCAB, EMBED_DIM)
    pooled = _pool_sc(r64, gidx)
    return _classifier_tc(pooled, W, b.reshape(1, -1))


# speedup vs baseline: 1.6242x; 1.6242x over previous
import functools

import jax
import jax.numpy as jnp
from jax import lax
from jax.experimental import pallas as pl
from jax.experimental.pallas import tpu as pltpu
from jax.experimental.pallas import tpu_sc as plsc

NC = 2
NS = 16
NW = NC * NS
LANES = 16

VOCAB = 1000000
BATCH = 4096
SEQ = 200
EMBED_DIM = 64

BPW = BATCH // NW
NV = EMBED_DIM // LANES
INV_SEQ = 1.0 / SEQ

BR = 2048
NBLK_IN = -(-VOCAB // BR)
NBLK_OUT = -(-NBLK_IN // 2)
RROWS = NBLK_OUT * BR

CHUNK = 128
NCHUNK_W = BPW * SEQ // CHUNK


def _repack_tc(table_t):
    def rep(lo_ref, hi_ref, o_ref):
        o_ref[...] = jnp.concatenate(
            [
                jnp.swapaxes(lo_ref[...], 0, 1),
                jnp.swapaxes(hi_ref[...], 0, 1),
            ],
            axis=1,
        )

    return pl.pallas_call(
        rep,
        grid=(NBLK_OUT,),
        in_specs=[
            pl.BlockSpec((EMBED_DIM, BR), lambda i: (0, 2 * i)),
            pl.BlockSpec(
                (EMBED_DIM, BR),
                lambda i: (0, jnp.minimum(2 * i + 1, NBLK_IN - 1)),
            ),
        ],
        out_specs=pl.BlockSpec((BR, 128), lambda i: (i, 0)),
        out_shape=jax.ShapeDtypeStruct((RROWS, 128), jnp.float32),
    )(table_t, table_t)


def _pool_sc(r64, gidx):
    mesh = plsc.VectorSubcoreMesh(core_axis_name="c", subcore_axis_name="s")

    @functools.partial(
        pl.kernel,
        mesh=mesh,
        compiler_params=pltpu.CompilerParams(use_tc_tiling_on_sc=False),
        out_type=jax.ShapeDtypeStruct((BATCH, EMBED_DIM), jnp.float32),
        scratch_types=[
            pltpu.VMEM((NCHUNK_W, CHUNK), jnp.int32),
            pltpu.VMEM((4, CHUNK, EMBED_DIM), jnp.float32),
            pltpu.VMEM((BPW, EMBED_DIM), jnp.float32),
            pltpu.SemaphoreType.DMA,
            pltpu.SemaphoreType.DMA,
            pltpu.SemaphoreType.DMA,
            pltpu.SemaphoreType.DMA,
        ],
    )
    def kb(r_hbm, idx_hbm, out_hbm, idx_v, rows_v, pooled_v,
           sem0, sem1, sem2, sem3):
        wid = lax.axis_index("s") * NC + lax.axis_index("c")
        cbase = wid * NCHUNK_W
        pltpu.sync_copy(idx_hbm.at[pl.ds(cbase, NCHUNK_W)], idx_v)

        def start(c, slot, sem):
            pltpu.async_copy(r_hbm.at[idx_v.at[c]], rows_v.at[slot], sem)

        def wait(c, slot, sem):
            pltpu.make_async_copy(
                r_hbm.at[idx_v.at[c]], rows_v.at[slot], sem
            ).wait()

        def reduce_chunk(c, slot, accs):
            s_raw = (SEQ - (CHUNK * c) % SEQ) % SEQ
            s = jnp.minimum(s_raw, CHUNK)
            flush = jnp.logical_and(s_raw < CHUNK,
                                    jnp.logical_or(c > 0, s_raw > 0))

            def red(rr, a):
                return tuple(
                    a[j] + rows_v[slot, rr, pl.ds(16 * j, 16)]
                    for j in range(NV)
                )

            accs = lax.fori_loop(jnp.int32(0), s, red, accs)

            @pl.when(flush)
            def _():
                rf = (CHUNK * c + s - 1) // SEQ
                for j in range(NV):
                    pooled_v[rf, pl.ds(16 * j, 16)] = accs[j] * INV_SEQ

            accs = tuple(
                jnp.where(flush, jnp.zeros((LANES,), jnp.float32), a)
                for a in accs
            )
            return lax.fori_loop(s, jnp.int32(CHUNK), red, accs)

        sems = (sem0, sem1, sem2, sem3)
        for ph in range(3):
            start(ph, ph, sems[ph])

        def body(cp, accs):
            c0 = 4 * cp
            for ph in range(4):
                c = c0 + ph
                nslot = (ph + 3) % 4

                @pl.when(c + 3 < NCHUNK_W)
                def _():
                    start(c + 3, nslot, sems[nslot])

                wait(c, ph, sems[ph])
                accs = reduce_chunk(c, ph, accs)
            return accs

        accs = lax.fori_loop(
            0, NCHUNK_W // 4, body,
            tuple(jnp.zeros((LANES,), jnp.float32) for _ in range(NV)),
        )
        for j in range(NV):
            pooled_v[BPW - 1, pl.ds(16 * j, 16)] = accs[j] * INV_SEQ
        pltpu.sync_copy(pooled_v, out_hbm.at[pl.ds(wid * BPW, BPW)])

    return kb(r64, gidx)


def _classifier_tc(pooled, W, b2):
    B, D = pooled.shape
    C = W.shape[0]
    BM = 512

    def mm(x_ref, w_ref, b_ref, o_ref):
        o_ref[...] = (
            lax.dot_general(
                x_ref[...],
                w_ref[...],
                (((1,), (1,)), ((), ())),
                preferred_element_type=jnp.float32,
            )
            + b_ref[...]
        )

    return pl.pallas_call(
        mm,
        grid=(B // BM,),
        in_specs=[
            pl.BlockSpec((BM, D), lambda i: (i, 0)),
            pl.BlockSpec((C, D), lambda i: (0, 0)),
            pl.BlockSpec((1, C), lambda i: (0, 0)),
        ],
        out_specs=pl.BlockSpec((BM, C), lambda i: (i, 0)),
        out_shape=jax.ShapeDtypeStruct((B, C), jnp.float32),
    )(pooled, W, b2)


def kernel(x_data, table, W, b):
    x = x_data.astype(jnp.int32)
    bb = x >> 11
    rr = x & (BR - 1)
    gidx = ((bb >> 1) << 12) + 2 * rr + (bb & 1)
    gidx = gidx.reshape(BATCH * SEQ // CHUNK, CHUNK)
    table_t = jnp.swapaxes(table, 0, 1)
    r = _repack_tc(table_t)
    r64 = r.reshape(2 * RROWS, EMBED_DIM)
    pooled = _pool_sc(r64, gidx)
    return _classifier_tc(pooled, W, b.reshape(1, -1))


# speedup vs baseline: 1.8832x; 1.1595x over previous
import functools

import jax
import jax.numpy as jnp
from jax import lax
from jax.experimental import pallas as pl
from jax.experimental.pallas import tpu as pltpu
from jax.experimental.pallas import tpu_sc as plsc

NC = 2
NS = 16
NW = NC * NS
LANES = 16

VOCAB = 1000000
BATCH = 4096
SEQ = 200
EMBED_DIM = 64

BPW = BATCH // NW
NV = EMBED_DIM // LANES
INV_SEQ = 1.0 / SEQ

BR = 4096
NBLK_IN = -(-VOCAB // BR)
NBLK_OUT = -(-NBLK_IN // 2)
RROWS = NBLK_OUT * BR

CHUNK = 128
NCHUNK_W = BPW * SEQ // CHUNK


def _repack_tc(table_t):
    def rep(lo_ref, hi_ref, o_ref):
        o_ref[...] = jnp.concatenate(
            [
                jnp.swapaxes(lo_ref[...], 0, 1),
                jnp.swapaxes(hi_ref[...], 0, 1),
            ],
            axis=1,
        )

    return pl.pallas_call(
        rep,
        grid=(NBLK_OUT,),
        in_specs=[
            pl.BlockSpec((EMBED_DIM, BR), lambda i: (0, 2 * i)),
            pl.BlockSpec(
                (EMBED_DIM, BR),
                lambda i: (0, jnp.minimum(2 * i + 1, NBLK_IN - 1)),
            ),
        ],
        out_specs=pl.BlockSpec((BR, 128), lambda i: (i, 0)),
        out_shape=jax.ShapeDtypeStruct((RROWS, 128), jnp.float32),
    )(table_t, table_t)


def _pool_sc(r64, gidx):
    mesh = plsc.VectorSubcoreMesh(core_axis_name="c", subcore_axis_name="s")

    @functools.partial(
        pl.kernel,
        mesh=mesh,
        compiler_params=pltpu.CompilerParams(use_tc_tiling_on_sc=False),
        out_type=jax.ShapeDtypeStruct((BATCH, EMBED_DIM), jnp.float32),
        scratch_types=[
            pltpu.VMEM((NCHUNK_W, CHUNK), jnp.int32),
            pltpu.VMEM((4, CHUNK, EMBED_DIM), jnp.float32),
            pltpu.VMEM((BPW, EMBED_DIM), jnp.float32),
            pltpu.SemaphoreType.DMA,
            pltpu.SemaphoreType.DMA,
            pltpu.SemaphoreType.DMA,
            pltpu.SemaphoreType.DMA,
        ],
    )
    def kb(r_hbm, idx_hbm, out_hbm, idx_v, rows_v, pooled_v,
           sem0, sem1, sem2, sem3):
        wid = lax.axis_index("s") * NC + lax.axis_index("c")
        cbase = wid * NCHUNK_W
        pltpu.sync_copy(idx_hbm.at[pl.ds(cbase, NCHUNK_W)], idx_v)

        def start(c, slot, sem):
            pltpu.async_copy(r_hbm.at[idx_v.at[c]], rows_v.at[slot], sem)

        def wait(c, slot, sem):
            pltpu.make_async_copy(
                r_hbm.at[idx_v.at[c]], rows_v.at[slot], sem
            ).wait()

        def reduce_chunk(c, slot, accs):
            s_raw = (SEQ - (CHUNK * c) % SEQ) % SEQ
            s = jnp.minimum(s_raw, CHUNK)
            flush = jnp.logical_and(s_raw < CHUNK,
                                    jnp.logical_or(c > 0, s_raw > 0))

            def red(rr, a):
                return tuple(
                    a[j] + rows_v[slot, rr, pl.ds(16 * j, 16)]
                    for j in range(NV)
                )

            accs = lax.fori_loop(jnp.int32(0), s, red, accs)

            @pl.when(flush)
            def _():
                rf = (CHUNK * c + s - 1) // SEQ
                for j in range(NV):
                    pooled_v[rf, pl.ds(16 * j, 16)] = accs[j] * INV_SEQ

            accs = tuple(
                jnp.where(flush, jnp.zeros((LANES,), jnp.float32), a)
                for a in accs
            )
            return lax.fori_loop(s, jnp.int32(CHUNK), red, accs)

        sems = (sem0, sem1, sem2, sem3)
        for ph in range(3):
            start(ph, ph, sems[ph])

        def body(cp, accs):
            c0 = 4 * cp
            for ph in range(4):
                c = c0 + ph
                nslot = (ph + 3) % 4

                @pl.when(c + 3 < NCHUNK_W)
                def _():
                    start(c + 3, nslot, sems[nslot])

                wait(c, ph, sems[ph])
                accs = reduce_chunk(c, ph, accs)
            return accs

        accs = lax.fori_loop(
            0, NCHUNK_W // 4, body,
            tuple(jnp.zeros((LANES,), jnp.float32) for _ in range(NV)),
        )
        for j in range(NV):
            pooled_v[BPW - 1, pl.ds(16 * j, 16)] = accs[j] * INV_SEQ
        pltpu.sync_copy(pooled_v, out_hbm.at[pl.ds(wid * BPW, BPW)])

    return kb(r64, gidx)


def _classifier_tc(pooled, W, b2):
    B, D = pooled.shape
    C = W.shape[0]
    BM = 512

    def mm(x_ref, w_ref, b_ref, o_ref):
        o_ref[...] = (
            lax.dot_general(
                x_ref[...],
                w_ref[...],
                (((1,), (1,)), ((), ())),
                preferred_element_type=jnp.float32,
            )
            + b_ref[...]
        )

    return pl.pallas_call(
        mm,
        grid=(B // BM,),
        in_specs=[
            pl.BlockSpec((BM, D), lambda i: (i, 0)),
            pl.BlockSpec((C, D), lambda i: (0, 0)),
            pl.BlockSpec((1, C), lambda i: (0, 0)),
        ],
        out_specs=pl.BlockSpec((BM, C), lambda i: (i, 0)),
        out_shape=jax.ShapeDtypeStruct((B, C), jnp.float32),
    )(pooled, W, b2)


def kernel(x_data, table, W, b):
    x = x_data.astype(jnp.int32)
    bb = x >> 12
    rr = x & (BR - 1)
    gidx = ((bb >> 1) << 13) + 2 * rr + (bb & 1)
    gidx = gidx.reshape(BATCH * SEQ // CHUNK, CHUNK)
    table_t = jnp.swapaxes(table, 0, 1)
    r = _repack_tc(table_t)
    r64 = r.reshape(2 * RROWS, EMBED_DIM)
    pooled = _pool_sc(r64, gidx)
    return _classifier_tc(pooled, W, b.reshape(1, -1))
